# Initial kernel scaffold; baseline (speedup 1.0000x reference)
import jax
import jax.numpy as jnp
from jax.experimental import pallas as pl


def kernel(encoded_tokens, pos_table):
    raise NotImplementedError("write your pallas kernel here")


# speedup vs baseline: 3.3776x; 3.3776x over previous
import jax
import jax.numpy as jnp
from jax.experimental import pallas as pl


_BN = 1024


def _add_kernel(enc_ref, pos_ref, out_ref):
    out_ref[0] = enc_ref[0] + pos_ref[...]


def kernel(encoded_tokens, pos_table):
    b, n, d = encoded_tokens.shape
    num_n = n // _BN
    return pl.pallas_call(
        _add_kernel,
        grid=(num_n, b),
        in_specs=[
            pl.BlockSpec((1, _BN, d), lambda i, j: (j, i, 0)),
            pl.BlockSpec((_BN, d), lambda i, j: (i, 0)),
        ],
        out_specs=pl.BlockSpec((1, _BN, d), lambda i, j: (j, i, 0)),
        out_shape=jax.ShapeDtypeStruct((b, n, d), encoded_tokens.dtype),
    )(encoded_tokens, pos_table)


# speedup vs baseline: 3.6180x; 1.0712x over previous
import jax
import jax.numpy as jnp
from jax.experimental import pallas as pl


_BN = 2048


def _add_kernel(enc_ref, pos_ref, out_ref):
    out_ref[0] = enc_ref[0] + pos_ref[...]


def kernel(encoded_tokens, pos_table):
    b, n, d = encoded_tokens.shape
    num_n = n // _BN
    return pl.pallas_call(
        _add_kernel,
        grid=(num_n, b),
        in_specs=[
            pl.BlockSpec((1, _BN, d), lambda i, j: (j, i, 0)),
            pl.BlockSpec((_BN, d), lambda i, j: (i, 0)),
        ],
        out_specs=pl.BlockSpec((1, _BN, d), lambda i, j: (j, i, 0)),
        out_shape=jax.ShapeDtypeStruct((b, n, d), encoded_tokens.dtype),
    )(encoded_tokens, pos_table)


# speedup vs baseline: 3.6274x; 1.0026x over previous
import jax
import jax.numpy as jnp
from jax.experimental import pallas as pl


_BN = 512


def _add_kernel(enc_ref, pos_ref, out_ref):
    out_ref[...] = enc_ref[...] + pos_ref[...]


def kernel(encoded_tokens, pos_table):
    b, n, d = encoded_tokens.shape
    num_n = n // _BN
    return pl.pallas_call(
        _add_kernel,
        grid=(num_n,),
        in_specs=[
            pl.BlockSpec((b, _BN, d), lambda i: (0, i, 0)),
            pl.BlockSpec((1, _BN, d), lambda i: (0, i, 0)),
        ],
        out_specs=pl.BlockSpec((b, _BN, d), lambda i: (0, i, 0)),
        out_shape=jax.ShapeDtypeStruct((b, n, d), encoded_tokens.dtype),
    )(encoded_tokens, pos_table[None])
